# baseline (speedup 1.0000x reference)
import functools

import jax
import jax.numpy as jnp
from jax import lax
from jax.experimental import pallas as pl
from jax.experimental.pallas import tpu as pltpu
from jax.experimental.pallas import tpu_sc as plsc

N = 100000
ND = 128
B = 2048
S1 = 16
S2 = 256
DT = 16
PD = 48
NW = 32
BPW = B // NW
PBLK = 2000
FBLK = 256


def _proj_body(x_ref, m_ref, o_ref):
    o_ref[...] = lax.dot_general(
        x_ref[...], m_ref[...], (((1,), (0,)), ((), ())),
        preferred_element_type=jnp.float32)


_proj_call = pl.pallas_call(
    _proj_body,
    grid=(N // PBLK,),
    in_specs=[
        pl.BlockSpec((PBLK, ND), lambda i: (i, 0)),
        pl.BlockSpec((ND, PD), lambda i: (0, 0)),
    ],
    out_specs=pl.BlockSpec((PBLK, PD), lambda i: (i, 0)),
    out_shape=jax.ShapeDtypeStruct((N, PD), jnp.float32),
)


def _sc_body(p_hbm, samples_hbm, beta1_hbm, con1_hbm, con0_hbm,
             idx_v, rows_v, b1_v, c1_v, c0_v, sem0, sem1, osem0, osem1):
    wid = lax.axis_index("s") * 2 + lax.axis_index("c")
    base = wid * BPW
    pltpu.sync_copy(samples_hbm.at[pl.ds(base, BPW)], idx_v)

    iota16 = lax.iota(jnp.int32, 16)
    c33 = jnp.full((16,), 33, jnp.int32)
    sems = (sem0, sem1)
    osems = (osem0, osem1)

    def softmax16(t):
        t = jnp.where(t >= 0.0, t, 0.01 * t)
        m = jnp.max(t)
        e = jnp.exp(t - m)
        return e / jnp.sum(e)

    def issue(i, slot):
        pltpu.async_copy(p_hbm.at[idx_v.at[i, pl.ds(0, 128)]],
                         rows_v.at[slot, pl.ds(0, 128)], sems[slot])
        pltpu.async_copy(p_hbm.at[idx_v.at[i, pl.ds(128, 128)]],
                         rows_v.at[slot, pl.ds(128, 128)], sems[slot])
        pltpu.async_copy(p_hbm.at[idx_v.at[i, pl.ds(256, 17)]],
                         rows_v.at[slot, pl.ds(256, 17)], sems[slot])

    def drain(slot):
        pltpu.make_async_copy(p_hbm.at[pl.ds(0, 273)],
                              rows_v.at[slot, pl.ds(0, 273)], sems[slot]).wait()

    def wait_out(slot):
        pltpu.make_async_copy(c0_v.at[slot], con0_hbm.at[base],
                              osems[slot]).wait()
        pltpu.make_async_copy(b1_v.at[slot], beta1_hbm.at[base],
                              osems[slot]).wait()
        pltpu.make_async_copy(c1_v.at[slot], con1_hbm.at[base],
                              osems[slot]).wait()

    def compute(i, slot):
        rv = rows_v.at[slot]

        a0 = rv[0, pl.ds(32, 16)][0]
        bv = plsc.load_gather(rv, [1 + iota16, c33])
        b0 = softmax16(a0 + bv)
        acc_a = jnp.zeros((16,), jnp.float32)
        acc_b = jnp.zeros((16,), jnp.float32)
        for dd in range(16):
            w = b0[dd]
            acc_a = acc_a + w * rv[1 + dd, pl.ds(0, 16)]
            acc_b = acc_b + w * rv[1 + dd, pl.ds(16, 16)]
        c0_v[slot, pl.ds(0, 16)] = rv[0, pl.ds(0, 16)]
        c0_v[slot, pl.ds(16, 16)] = rv[0, pl.ds(16, 16)]
        c0_v[slot, pl.ds(32, 16)] = acc_a
        c0_v[slot, pl.ds(48, 16)] = acc_b

        for s in range(S1):
            a_t = rv[1 + s, pl.ds(32, 16)][0]
            base_r = 17 + 16 * s
            bvs = plsc.load_gather(rv, [base_r + iota16, c33])
            bb = softmax16(a_t + bvs)
            b1_v[slot, s, :] = bb
            aa = jnp.zeros((16,), jnp.float32)
            ab = jnp.zeros((16,), jnp.float32)
            for dd in range(16):
                w = bb[dd]
                aa = aa + w * rv[base_r + dd, pl.ds(0, 16)]
                ab = ab + w * rv[base_r + dd, pl.ds(16, 16)]
            c1_v[slot, s, pl.ds(0, 16)] = rv[1 + s, pl.ds(0, 16)]
            c1_v[slot, s, pl.ds(16, 16)] = rv[1 + s, pl.ds(16, 16)]
            c1_v[slot, s, pl.ds(32, 16)] = aa
            c1_v[slot, s, pl.ds(48, 16)] = ab

        pltpu.async_copy(c0_v.at[slot], con0_hbm.at[base + i], osems[slot])
        pltpu.async_copy(b1_v.at[slot], beta1_hbm.at[base + i], osems[slot])
        pltpu.async_copy(c1_v.at[slot], con1_hbm.at[base + i], osems[slot])

    issue(0, 0)

    def gbody(g, carry):
        i = 2 * g
        issue(i + 1, 1)
        drain(0)

        @pl.when(g > 0)
        def _():
            wait_out(0)

        compute(i, 0)

        @pl.when(i + 2 < BPW)
        def _():
            issue(i + 2, 0)

        drain(1)

        @pl.when(g > 0)
        def _():
            wait_out(1)

        compute(i + 1, 1)
        return carry

    lax.fori_loop(0, BPW // 2, gbody, 0)
    wait_out(0)
    wait_out(1)


_sc_call = functools.partial(
    pl.kernel,
    mesh=plsc.VectorSubcoreMesh(core_axis_name="c", subcore_axis_name="s"),
    compiler_params=pltpu.CompilerParams(
        needs_layout_passes=False, use_tc_tiling_on_sc=False),
    out_type=(
        jax.ShapeDtypeStruct((B, S1, DT), jnp.float32),
        jax.ShapeDtypeStruct((B, S1, 64), jnp.float32),
        jax.ShapeDtypeStruct((B, 64), jnp.float32),
    ),
    scratch_types=[
        pltpu.VMEM((BPW, 273), jnp.int32),
        pltpu.VMEM((2, 280, PD), jnp.float32),
        pltpu.VMEM((2, S1, DT), jnp.float32),
        pltpu.VMEM((2, S1, 64), jnp.float32),
        pltpu.VMEM((2, 64), jnp.float32),
        pltpu.SemaphoreType.DMA,
        pltpu.SemaphoreType.DMA,
        pltpu.SemaphoreType.DMA,
        pltpu.SemaphoreType.DMA,
    ],
)(_sc_body)


def _fin_body(c1_ref, c0_ref, w_ref, v0_ref, q_ref, bh_ref, fh_ref):
    W = w_ref[...]
    V0 = v0_ref[...]

    c1f = c1_ref[...].reshape(FBLK * S1, 64)
    z1 = lax.dot_general(c1f, W, (((1,), (1,)), ((), ())),
                         preferred_element_type=jnp.float32)
    h1 = 1.0 / (1.0 + jnp.exp(-z1))
    g1 = lax.dot_general(h1, V0, (((1,), (1,)), ((), ())),
                         preferred_element_type=jnp.float32)
    q = q_ref[...]
    d = lax.dot_general(h1, q[:, 1:2], (((1,), (0,)), ((), ())),
                        preferred_element_type=jnp.float32)
    d3 = d.reshape(FBLK, S1, 1)

    z0 = lax.dot_general(c0_ref[...], W, (((1,), (1,)), ((), ())),
                         preferred_element_type=jnp.float32)
    h0 = 1.0 / (1.0 + jnp.exp(-z0))
    g0 = lax.dot_general(h0, V0, (((1,), (1,)), ((), ())),
                         preferred_element_type=jnp.float32)
    gam = lax.dot_general(h0, q[:, 0:1], (((1,), (0,)), ((), ())),
                          preferred_element_type=jnp.float32)
    g3 = gam.reshape(FBLK, 1, 1)

    t = g3 + d3
    t = jnp.where(t >= 0.0, t, 0.01 * t)
    m = jnp.max(t, axis=1, keepdims=True)
    e = jnp.exp(t - m)
    bh = e / jnp.sum(e, axis=1, keepdims=True)
    bh_ref[...] = bh

    g13 = g1.reshape(FBLK, S1, 32)
    fh_ref[...] = jnp.sum(g13 * bh, axis=1, keepdims=True)


_fin_call = pl.pallas_call(
    _fin_body,
    grid=(B // FBLK,),
    in_specs=[
        pl.BlockSpec((FBLK, S1, 64), lambda i: (i, 0, 0)),
        pl.BlockSpec((FBLK, 64), lambda i: (i, 0)),
        pl.BlockSpec((32, 64), lambda i: (0, 0)),
        pl.BlockSpec((32, 32), lambda i: (0, 0)),
        pl.BlockSpec((32, 2), lambda i: (0, 0)),
    ],
    out_specs=[
        pl.BlockSpec((FBLK, S1, 1), lambda i: (i, 0, 0)),
        pl.BlockSpec((FBLK, 1, 32), lambda i: (i, 0, 0)),
    ],
    out_shape=[
        jax.ShapeDtypeStruct((B, S1, 1), jnp.float32),
        jax.ShapeDtypeStruct((B, 1, 32), jnp.float32),
    ],
)


def kernel(x, samples, V1_h0, w1_h0, V1_h1_att, w1_h1, V1_h1, weights_hops_1):
    a_col = V1_h1_att.T @ w1_h1[:32]
    b_col = V1_h1_att.T @ w1_h1[32:]
    M = jnp.concatenate(
        [V1_h1.T, a_col[:, None], b_col[:, None],
         jnp.zeros((ND, PD - 34), jnp.float32)], axis=1)
    q = jnp.stack([V1_h0.T @ w1_h0[:32], V1_h0.T @ w1_h0[32:]], axis=1)

    P = _proj_call(x, M)
    beta1, con1, con0 = _sc_call(P, samples)
    bh, fh = _fin_call(con1, con0, weights_hops_1, V1_h0, q)

    out1 = fh[:, 0, :].T
    beta_step = jnp.concatenate([bh, beta1], axis=2)
    return out1, beta_step


# speedup vs baseline: 1.1001x; 1.1001x over previous
import functools

import jax
import jax.numpy as jnp
from jax import lax
from jax.experimental import pallas as pl
from jax.experimental.pallas import tpu as pltpu
from jax.experimental.pallas import tpu_sc as plsc

N = 100000
ND = 128
B = 2048
S1 = 16
S2 = 256
DT = 16
PD = 48
NW = 32
BPW = B // NW
PBLK = 2000
FBLK = 256


def _proj_body(x_ref, m_ref, o_ref):
    o_ref[...] = lax.dot_general(
        x_ref[...], m_ref[...], (((1,), (0,)), ((), ())),
        preferred_element_type=jnp.float32)


_proj_call = pl.pallas_call(
    _proj_body,
    grid=(N // PBLK,),
    in_specs=[
        pl.BlockSpec((PBLK, ND), lambda i: (i, 0)),
        pl.BlockSpec((ND, PD), lambda i: (0, 0)),
    ],
    out_specs=pl.BlockSpec((PBLK, PD), lambda i: (i, 0)),
    out_shape=jax.ShapeDtypeStruct((N, PD), jnp.float32),
)


def _sc_body(p_hbm, samples_hbm, beta1_hbm, con1_hbm, con0_hbm,
             idx_v, rows_v, b1_v, c1_v, c0_v, sem0, sem1, osem0, osem1):
    wid = lax.axis_index("s") * 2 + lax.axis_index("c")
    base = wid * BPW
    pltpu.sync_copy(samples_hbm.at[pl.ds(base, BPW)], idx_v)

    iota16 = lax.iota(jnp.int32, 16)
    c33 = jnp.full((16,), 33, jnp.int32)
    sems = (sem0, sem1)
    osems = (osem0, osem1)

    def softmax16(t):
        t = jnp.where(t >= 0.0, t, 0.01 * t)
        m = jnp.max(t)
        e = jnp.exp(t - m)
        return e / jnp.sum(e)

    def issue(i, slot):
        pltpu.async_copy(p_hbm.at[idx_v.at[i, pl.ds(0, 128)]],
                         rows_v.at[slot, pl.ds(0, 128)], sems[slot])
        pltpu.async_copy(p_hbm.at[idx_v.at[i, pl.ds(128, 128)]],
                         rows_v.at[slot, pl.ds(128, 128)], sems[slot])
        pltpu.async_copy(p_hbm.at[idx_v.at[i, pl.ds(256, 17)]],
                         rows_v.at[slot, pl.ds(256, 17)], sems[slot])

    def drain(slot):
        pltpu.make_async_copy(p_hbm.at[pl.ds(0, 273)],
                              rows_v.at[slot, pl.ds(0, 273)], sems[slot]).wait()

    def wait_out(slot):
        pltpu.make_async_copy(c0_v.at[slot], con0_hbm.at[base],
                              osems[slot]).wait()
        pltpu.make_async_copy(b1_v.at[slot], beta1_hbm.at[base],
                              osems[slot]).wait()
        pltpu.make_async_copy(c1_v.at[slot], con1_hbm.at[base],
                              osems[slot]).wait()

    def compute(i, slot):
        rv = rows_v.at[slot]

        a0 = rv[0, pl.ds(32, 16)][0]
        bv = plsc.load_gather(rv, [1 + iota16, c33])
        b0 = softmax16(a0 + bv)
        acc_a = jnp.zeros((16,), jnp.float32)
        acc_b = jnp.zeros((16,), jnp.float32)
        for dd in range(16):
            w = b0[dd]
            acc_a = acc_a + w * rv[1 + dd, pl.ds(0, 16)]
            acc_b = acc_b + w * rv[1 + dd, pl.ds(16, 16)]
        c0_v[slot, pl.ds(0, 16)] = rv[0, pl.ds(0, 16)]
        c0_v[slot, pl.ds(16, 16)] = rv[0, pl.ds(16, 16)]
        c0_v[slot, pl.ds(32, 16)] = acc_a
        c0_v[slot, pl.ds(48, 16)] = acc_b

        def sbody(s, c):
            a_t = rv[1 + s, pl.ds(32, 16)][0]
            base_r = 17 + 16 * s
            bvs = plsc.load_gather(rv, [base_r + iota16, c33])
            bb = softmax16(a_t + bvs)
            b1_v[slot, s, :] = bb
            aa = jnp.zeros((16,), jnp.float32)
            ab = jnp.zeros((16,), jnp.float32)
            for dd in range(16):
                w = bb[dd]
                aa = aa + w * rv[base_r + dd, pl.ds(0, 16)]
                ab = ab + w * rv[base_r + dd, pl.ds(16, 16)]
            c1_v[slot, s, pl.ds(0, 16)] = rv[1 + s, pl.ds(0, 16)]
            c1_v[slot, s, pl.ds(16, 16)] = rv[1 + s, pl.ds(16, 16)]
            c1_v[slot, s, pl.ds(32, 16)] = aa
            c1_v[slot, s, pl.ds(48, 16)] = ab
            return c

        lax.fori_loop(0, S1, sbody, 0)

        pltpu.async_copy(c0_v.at[slot], con0_hbm.at[base + i], osems[slot])
        pltpu.async_copy(b1_v.at[slot], beta1_hbm.at[base + i], osems[slot])
        pltpu.async_copy(c1_v.at[slot], con1_hbm.at[base + i], osems[slot])

    issue(0, 0)

    def gbody(g, carry):
        i = 2 * g
        issue(i + 1, 1)
        drain(0)

        @pl.when(g > 0)
        def _():
            wait_out(0)

        compute(i, 0)

        @pl.when(i + 2 < BPW)
        def _():
            issue(i + 2, 0)

        drain(1)

        @pl.when(g > 0)
        def _():
            wait_out(1)

        compute(i + 1, 1)
        return carry

    lax.fori_loop(0, BPW // 2, gbody, 0)
    wait_out(0)
    wait_out(1)


_sc_call = functools.partial(
    pl.kernel,
    mesh=plsc.VectorSubcoreMesh(core_axis_name="c", subcore_axis_name="s"),
    compiler_params=pltpu.CompilerParams(
        needs_layout_passes=False, use_tc_tiling_on_sc=False),
    out_type=(
        jax.ShapeDtypeStruct((B, S1, DT), jnp.float32),
        jax.ShapeDtypeStruct((B, S1, 64), jnp.float32),
        jax.ShapeDtypeStruct((B, 64), jnp.float32),
    ),
    scratch_types=[
        pltpu.VMEM((BPW, 273), jnp.int32),
        pltpu.VMEM((2, 280, PD), jnp.float32),
        pltpu.VMEM((2, S1, DT), jnp.float32),
        pltpu.VMEM((2, S1, 64), jnp.float32),
        pltpu.VMEM((2, 64), jnp.float32),
        pltpu.SemaphoreType.DMA,
        pltpu.SemaphoreType.DMA,
        pltpu.SemaphoreType.DMA,
        pltpu.SemaphoreType.DMA,
    ],
)(_sc_body)


def _fin_body(c1_ref, c0_ref, w_ref, v0_ref, q_ref, bh_ref, fh_ref):
    W = w_ref[...]
    V0 = v0_ref[...]

    c1f = c1_ref[...].reshape(FBLK * S1, 64)
    z1 = lax.dot_general(c1f, W, (((1,), (1,)), ((), ())),
                         preferred_element_type=jnp.float32)
    h1 = 1.0 / (1.0 + jnp.exp(-z1))
    g1 = lax.dot_general(h1, V0, (((1,), (1,)), ((), ())),
                         preferred_element_type=jnp.float32)
    q = q_ref[...]
    d = lax.dot_general(h1, q[:, 1:2], (((1,), (0,)), ((), ())),
                        preferred_element_type=jnp.float32)
    d3 = d.reshape(FBLK, S1, 1)

    z0 = lax.dot_general(c0_ref[...], W, (((1,), (1,)), ((), ())),
                         preferred_element_type=jnp.float32)
    h0 = 1.0 / (1.0 + jnp.exp(-z0))
    g0 = lax.dot_general(h0, V0, (((1,), (1,)), ((), ())),
                         preferred_element_type=jnp.float32)
    gam = lax.dot_general(h0, q[:, 0:1], (((1,), (0,)), ((), ())),
                          preferred_element_type=jnp.float32)
    g3 = gam.reshape(FBLK, 1, 1)

    t = g3 + d3
    t = jnp.where(t >= 0.0, t, 0.01 * t)
    m = jnp.max(t, axis=1, keepdims=True)
    e = jnp.exp(t - m)
    bh = e / jnp.sum(e, axis=1, keepdims=True)
    bh_ref[...] = bh

    g13 = g1.reshape(FBLK, S1, 32)
    fh_ref[...] = jnp.sum(g13 * bh, axis=1, keepdims=True)


_fin_call = pl.pallas_call(
    _fin_body,
    grid=(B // FBLK,),
    in_specs=[
        pl.BlockSpec((FBLK, S1, 64), lambda i: (i, 0, 0)),
        pl.BlockSpec((FBLK, 64), lambda i: (i, 0)),
        pl.BlockSpec((32, 64), lambda i: (0, 0)),
        pl.BlockSpec((32, 32), lambda i: (0, 0)),
        pl.BlockSpec((32, 2), lambda i: (0, 0)),
    ],
    out_specs=[
        pl.BlockSpec((FBLK, S1, 1), lambda i: (i, 0, 0)),
        pl.BlockSpec((FBLK, 1, 32), lambda i: (i, 0, 0)),
    ],
    out_shape=[
        jax.ShapeDtypeStruct((B, S1, 1), jnp.float32),
        jax.ShapeDtypeStruct((B, 1, 32), jnp.float32),
    ],
)


def kernel(x, samples, V1_h0, w1_h0, V1_h1_att, w1_h1, V1_h1, weights_hops_1):
    a_col = V1_h1_att.T @ w1_h1[:32]
    b_col = V1_h1_att.T @ w1_h1[32:]
    M = jnp.concatenate(
        [V1_h1.T, a_col[:, None], b_col[:, None],
         jnp.zeros((ND, PD - 34), jnp.float32)], axis=1)
    q = jnp.stack([V1_h0.T @ w1_h0[:32], V1_h0.T @ w1_h0[32:]], axis=1)

    P = _proj_call(x, M)
    beta1, con1, con0 = _sc_call(P, samples)
    bh, fh = _fin_call(con1, con0, weights_hops_1, V1_h0, q)

    out1 = fh[:, 0, :].T
    beta_step = jnp.concatenate([bh, beta1], axis=2)
    return out1, beta_step


# speedup vs baseline: 1.1937x; 1.0851x over previous
import functools

import jax
import jax.numpy as jnp
from jax import lax
from jax.experimental import pallas as pl
from jax.experimental.pallas import tpu as pltpu
from jax.experimental.pallas import tpu_sc as plsc

N = 100000
ND = 128
B = 2048
S1 = 16
S2 = 256
DT = 16
PD = 128
NW = 32
BPW = B // NW
PBLK = 2000
FBLK = 256


def _proj_body(x_ref, m_ref, o_ref):
    o_ref[...] = lax.dot_general(
        x_ref[...], m_ref[...], (((1,), (0,)), ((), ())),
        preferred_element_type=jnp.float32)


_proj_call = pl.pallas_call(
    _proj_body,
    grid=(N // PBLK,),
    in_specs=[
        pl.BlockSpec((PBLK, ND), lambda i: (i, 0)),
        pl.BlockSpec((ND, PD), lambda i: (0, 0)),
    ],
    out_specs=pl.BlockSpec((PBLK, PD), lambda i: (i, 0)),
    out_shape=jax.ShapeDtypeStruct((N, PD), jnp.float32),
)


def _sc_body(p_hbm, samples_hbm, beta1_hbm, con1_hbm, con0_hbm,
             idx_v, rows_v, b1_v, c1_v, c0_v, sem0, sem1, osem0, osem1):
    wid = lax.axis_index("s") * 2 + lax.axis_index("c")
    base = wid * BPW
    pltpu.sync_copy(samples_hbm.at[pl.ds(base, BPW)], idx_v)

    iota16 = lax.iota(jnp.int32, 16)
    c33 = jnp.full((16,), 33, jnp.int32)
    sems = (sem0, sem1)
    osems = (osem0, osem1)

    def softmax16(t):
        t = jnp.where(t >= 0.0, t, 0.01 * t)
        m = jnp.max(t)
        e = jnp.exp(t - m)
        return e / jnp.sum(e)

    def issue(i, slot):
        pltpu.async_copy(p_hbm.at[idx_v.at[i, pl.ds(0, 128)]],
                         rows_v.at[slot, pl.ds(0, 128)], sems[slot])
        pltpu.async_copy(p_hbm.at[idx_v.at[i, pl.ds(128, 128)]],
                         rows_v.at[slot, pl.ds(128, 128)], sems[slot])
        pltpu.async_copy(p_hbm.at[idx_v.at[i, pl.ds(256, 17)]],
                         rows_v.at[slot, pl.ds(256, 17)], sems[slot])

    def drain(slot):
        pltpu.make_async_copy(p_hbm.at[pl.ds(0, 273)],
                              rows_v.at[slot, pl.ds(0, 273)], sems[slot]).wait()

    def wait_out(slot):
        pltpu.make_async_copy(c0_v.at[slot], con0_hbm.at[base],
                              osems[slot]).wait()
        pltpu.make_async_copy(b1_v.at[slot], beta1_hbm.at[base],
                              osems[slot]).wait()
        pltpu.make_async_copy(c1_v.at[slot], con1_hbm.at[base],
                              osems[slot]).wait()

    def compute(i, slot):
        rv = rows_v.at[slot]

        a0 = rv[0, pl.ds(32, 16)][0]
        bv = plsc.load_gather(rv, [1 + iota16, c33])
        b0 = softmax16(a0 + bv)
        acc_a = jnp.zeros((16,), jnp.float32)
        acc_b = jnp.zeros((16,), jnp.float32)
        for dd in range(16):
            w = b0[dd]
            acc_a = acc_a + w * rv[1 + dd, pl.ds(0, 16)]
            acc_b = acc_b + w * rv[1 + dd, pl.ds(16, 16)]
        c0_v[slot, pl.ds(0, 16)] = rv[0, pl.ds(0, 16)]
        c0_v[slot, pl.ds(16, 16)] = rv[0, pl.ds(16, 16)]
        c0_v[slot, pl.ds(32, 16)] = acc_a
        c0_v[slot, pl.ds(48, 16)] = acc_b

        def sbody(s, c):
            a_t = rv[1 + s, pl.ds(32, 16)][0]
            base_r = 17 + 16 * s
            bvs = plsc.load_gather(rv, [base_r + iota16, c33])
            bb = softmax16(a_t + bvs)
            b1_v[slot, s, :] = bb
            aa = jnp.zeros((16,), jnp.float32)
            ab = jnp.zeros((16,), jnp.float32)
            for dd in range(16):
                w = bb[dd]
                aa = aa + w * rv[base_r + dd, pl.ds(0, 16)]
                ab = ab + w * rv[base_r + dd, pl.ds(16, 16)]
            c1_v[slot, s, pl.ds(0, 16)] = rv[1 + s, pl.ds(0, 16)]
            c1_v[slot, s, pl.ds(16, 16)] = rv[1 + s, pl.ds(16, 16)]
            c1_v[slot, s, pl.ds(32, 16)] = aa
            c1_v[slot, s, pl.ds(48, 16)] = ab
            return c

        lax.fori_loop(0, S1, sbody, 0)

        pltpu.async_copy(c0_v.at[slot], con0_hbm.at[base + i], osems[slot])
        pltpu.async_copy(b1_v.at[slot], beta1_hbm.at[base + i], osems[slot])
        pltpu.async_copy(c1_v.at[slot], con1_hbm.at[base + i], osems[slot])

    issue(0, 0)

    def gbody(g, carry):
        i = 2 * g
        issue(i + 1, 1)
        drain(0)

        @pl.when(g > 0)
        def _():
            wait_out(0)

        compute(i, 0)

        @pl.when(i + 2 < BPW)
        def _():
            issue(i + 2, 0)

        drain(1)

        @pl.when(g > 0)
        def _():
            wait_out(1)

        compute(i + 1, 1)
        return carry

    lax.fori_loop(0, BPW // 2, gbody, 0)
    wait_out(0)
    wait_out(1)


_sc_call = functools.partial(
    pl.kernel,
    mesh=plsc.VectorSubcoreMesh(core_axis_name="c", subcore_axis_name="s"),
    compiler_params=pltpu.CompilerParams(
        needs_layout_passes=False, use_tc_tiling_on_sc=False),
    out_type=(
        jax.ShapeDtypeStruct((B, S1, DT), jnp.float32),
        jax.ShapeDtypeStruct((B, S1, 128), jnp.float32),
        jax.ShapeDtypeStruct((B, 128), jnp.float32),
    ),
    scratch_types=[
        pltpu.VMEM((BPW, 273), jnp.int32),
        pltpu.VMEM((2, 280, PD), jnp.float32),
        pltpu.VMEM((2, S1, DT), jnp.float32),
        pltpu.VMEM((2, S1, 128), jnp.float32),
        pltpu.VMEM((2, 128), jnp.float32),
        pltpu.SemaphoreType.DMA,
        pltpu.SemaphoreType.DMA,
        pltpu.SemaphoreType.DMA,
        pltpu.SemaphoreType.DMA,
    ],
)(_sc_body)


def _fin_body(c1_ref, c0_ref, w_ref, v0_ref, q_ref, bh_ref, fh_ref):
    W = w_ref[...]
    V0 = v0_ref[...]

    c1f = c1_ref[:, :, :64].reshape(FBLK * S1, 64)
    z1 = lax.dot_general(c1f, W, (((1,), (1,)), ((), ())),
                         preferred_element_type=jnp.float32)
    h1 = 1.0 / (1.0 + jnp.exp(-z1))
    g1 = lax.dot_general(h1, V0, (((1,), (1,)), ((), ())),
                         preferred_element_type=jnp.float32)
    q = q_ref[...]
    d = lax.dot_general(h1, q[:, 1:2], (((1,), (0,)), ((), ())),
                        preferred_element_type=jnp.float32)
    d3 = d.reshape(FBLK, S1, 1)

    z0 = lax.dot_general(c0_ref[:, :64], W, (((1,), (1,)), ((), ())),
                         preferred_element_type=jnp.float32)
    h0 = 1.0 / (1.0 + jnp.exp(-z0))
    g0 = lax.dot_general(h0, V0, (((1,), (1,)), ((), ())),
                         preferred_element_type=jnp.float32)
    gam = lax.dot_general(h0, q[:, 0:1], (((1,), (0,)), ((), ())),
                          preferred_element_type=jnp.float32)
    g3 = gam.reshape(FBLK, 1, 1)

    t = g3 + d3
    t = jnp.where(t >= 0.0, t, 0.01 * t)
    m = jnp.max(t, axis=1, keepdims=True)
    e = jnp.exp(t - m)
    bh = e / jnp.sum(e, axis=1, keepdims=True)
    bh_ref[...] = bh

    g13 = g1.reshape(FBLK, S1, 32)
    fh_ref[...] = jnp.sum(g13 * bh, axis=1, keepdims=True)


_fin_call = pl.pallas_call(
    _fin_body,
    grid=(B // FBLK,),
    in_specs=[
        pl.BlockSpec((FBLK, S1, 128), lambda i: (i, 0, 0)),
        pl.BlockSpec((FBLK, 128), lambda i: (i, 0)),
        pl.BlockSpec((32, 64), lambda i: (0, 0)),
        pl.BlockSpec((32, 32), lambda i: (0, 0)),
        pl.BlockSpec((32, 2), lambda i: (0, 0)),
    ],
    out_specs=[
        pl.BlockSpec((FBLK, S1, 1), lambda i: (i, 0, 0)),
        pl.BlockSpec((FBLK, 1, 32), lambda i: (i, 0, 0)),
    ],
    out_shape=[
        jax.ShapeDtypeStruct((B, S1, 1), jnp.float32),
        jax.ShapeDtypeStruct((B, 1, 32), jnp.float32),
    ],
)


def kernel(x, samples, V1_h0, w1_h0, V1_h1_att, w1_h1, V1_h1, weights_hops_1):
    a_col = V1_h1_att.T @ w1_h1[:32]
    b_col = V1_h1_att.T @ w1_h1[32:]
    M = jnp.concatenate(
        [V1_h1.T, a_col[:, None], b_col[:, None],
         jnp.zeros((ND, PD - 34), jnp.float32)], axis=1)
    q = jnp.stack([V1_h0.T @ w1_h0[:32], V1_h0.T @ w1_h0[32:]], axis=1)

    P = _proj_call(x, M)
    beta1, con1, con0 = _sc_call(P, samples)
    bh, fh = _fin_call(con1, con0, weights_hops_1, V1_h0, q)

    out1 = fh[:, 0, :].T
    beta_step = jnp.concatenate([bh, beta1], axis=2)
    return out1, beta_step


# speedup vs baseline: 1.2628x; 1.0579x over previous
import functools

import jax
import jax.numpy as jnp
from jax import lax
from jax.experimental import pallas as pl
from jax.experimental.pallas import tpu as pltpu
from jax.experimental.pallas import tpu_sc as plsc

N = 100000
ND = 128
B = 2048
S1 = 16
S2 = 256
DT = 16
PD = 128
NW = 32
BPW = B // NW
PBLK = 2000
FBLK = 256


def _proj_body(x_ref, m_ref, o_ref):
    o_ref[...] = lax.dot_general(
        x_ref[...], m_ref[...], (((1,), (0,)), ((), ())),
        preferred_element_type=jnp.float32)


_proj_call = pl.pallas_call(
    _proj_body,
    grid=(N // PBLK,),
    in_specs=[
        pl.BlockSpec((PBLK, ND), lambda i: (i, 0)),
        pl.BlockSpec((ND, PD), lambda i: (0, 0)),
    ],
    out_specs=pl.BlockSpec((PBLK, PD), lambda i: (i, 0)),
    out_shape=jax.ShapeDtypeStruct((N, PD), jnp.float32),
)


def _sc_body(p_hbm, samples_hbm, beta1_hbm, con1_hbm, con0_hbm,
             idx_v, rows_v, b1_v, c1_v, c0_v, sem0, sem1, osem0, osem1):
    wid = lax.axis_index("s") * 2 + lax.axis_index("c")
    base = wid * BPW
    pltpu.sync_copy(samples_hbm.at[pl.ds(base, BPW)], idx_v)

    iota16 = lax.iota(jnp.int32, 16)
    c33 = jnp.full((16,), 33, jnp.int32)
    sems = (sem0, sem1)
    osems = (osem0, osem1)

    def softmax16(t):
        t = jnp.where(t >= 0.0, t, 0.01 * t)
        m = jnp.max(t)
        e = jnp.exp(t - m)
        return e / jnp.sum(e)

    def issue(i, slot):
        pltpu.async_copy(p_hbm.at[idx_v.at[i, pl.ds(0, 128)]],
                         rows_v.at[slot, pl.ds(0, 128)], sems[slot])
        pltpu.async_copy(p_hbm.at[idx_v.at[i, pl.ds(128, 128)]],
                         rows_v.at[slot, pl.ds(128, 128)], sems[slot])
        pltpu.async_copy(p_hbm.at[idx_v.at[i, pl.ds(256, 17)]],
                         rows_v.at[slot, pl.ds(256, 17)], sems[slot])

    def drain(slot):
        pltpu.make_async_copy(p_hbm.at[pl.ds(0, 273)],
                              rows_v.at[slot, pl.ds(0, 273)], sems[slot]).wait()

    def wait_out(slot):
        pltpu.make_async_copy(c0_v.at[slot], con0_hbm.at[base],
                              osems[slot]).wait()
        pltpu.make_async_copy(b1_v.at[slot], beta1_hbm.at[base],
                              osems[slot]).wait()
        pltpu.make_async_copy(c1_v.at[slot], con1_hbm.at[base],
                              osems[slot]).wait()

    def compute(i, slot):
        rv = rows_v.at[slot]

        a0 = rv[0, pl.ds(32, 16)][0]
        bv = plsc.load_gather(rv, [1 + iota16, c33])
        b0 = softmax16(a0 + bv)
        acc_a = jnp.zeros((16,), jnp.float32)
        acc_b = jnp.zeros((16,), jnp.float32)
        for dd in range(16):
            w = b0[dd]
            acc_a = acc_a + w * rv[1 + dd, pl.ds(0, 16)]
            acc_b = acc_b + w * rv[1 + dd, pl.ds(16, 16)]
        c0_v[slot, pl.ds(0, 16)] = rv[0, pl.ds(0, 16)]
        c0_v[slot, pl.ds(16, 16)] = rv[0, pl.ds(16, 16)]
        c0_v[slot, pl.ds(32, 16)] = acc_a
        c0_v[slot, pl.ds(48, 16)] = acc_b

        def sbody(s, c):
            a_t = rv[1 + s, pl.ds(32, 16)][0]
            base_r = 17 + 16 * s
            bvs = plsc.load_gather(rv, [base_r + iota16, c33])
            bb = softmax16(a_t + bvs)
            b1_v[slot, s, pl.ds(0, 16)] = bb
            aa = jnp.zeros((16,), jnp.float32)
            ab = jnp.zeros((16,), jnp.float32)
            for dd in range(16):
                w = bb[dd]
                aa = aa + w * rv[base_r + dd, pl.ds(0, 16)]
                ab = ab + w * rv[base_r + dd, pl.ds(16, 16)]
            c1_v[slot, s, pl.ds(0, 16)] = rv[1 + s, pl.ds(0, 16)]
            c1_v[slot, s, pl.ds(16, 16)] = rv[1 + s, pl.ds(16, 16)]
            c1_v[slot, s, pl.ds(32, 16)] = aa
            c1_v[slot, s, pl.ds(48, 16)] = ab
            return c

        lax.fori_loop(0, S1, sbody, 0)

        pltpu.async_copy(c0_v.at[slot], con0_hbm.at[base + i], osems[slot])
        pltpu.async_copy(b1_v.at[slot], beta1_hbm.at[base + i], osems[slot])
        pltpu.async_copy(c1_v.at[slot], con1_hbm.at[base + i], osems[slot])

    issue(0, 0)

    def gbody(g, carry):
        i = 2 * g
        issue(i + 1, 1)
        drain(0)

        @pl.when(g > 0)
        def _():
            wait_out(0)

        compute(i, 0)

        @pl.when(i + 2 < BPW)
        def _():
            issue(i + 2, 0)

        drain(1)

        @pl.when(g > 0)
        def _():
            wait_out(1)

        compute(i + 1, 1)
        return carry

    lax.fori_loop(0, BPW // 2, gbody, 0)
    wait_out(0)
    wait_out(1)


_sc_call = functools.partial(
    pl.kernel,
    mesh=plsc.VectorSubcoreMesh(core_axis_name="c", subcore_axis_name="s"),
    compiler_params=pltpu.CompilerParams(
        needs_layout_passes=False, use_tc_tiling_on_sc=False),
    out_type=(
        jax.ShapeDtypeStruct((B, S1, 128), jnp.float32),
        jax.ShapeDtypeStruct((B, S1, 128), jnp.float32),
        jax.ShapeDtypeStruct((B, 128), jnp.float32),
    ),
    scratch_types=[
        pltpu.VMEM((BPW, 273), jnp.int32),
        pltpu.VMEM((2, 280, PD), jnp.float32),
        pltpu.VMEM((2, S1, 128), jnp.float32),
        pltpu.VMEM((2, S1, 128), jnp.float32),
        pltpu.VMEM((2, 128), jnp.float32),
        pltpu.SemaphoreType.DMA,
        pltpu.SemaphoreType.DMA,
        pltpu.SemaphoreType.DMA,
        pltpu.SemaphoreType.DMA,
    ],
)(_sc_body)


def _fin_body(c1_ref, c0_ref, b1_ref, w_ref, v0_ref, q_ref, bs_ref, o1_ref):
    W = w_ref[...]
    V0 = v0_ref[...]

    c1f = c1_ref[:, :, :64].reshape(FBLK * S1, 64)
    z1 = lax.dot_general(c1f, W, (((1,), (1,)), ((), ())),
                         preferred_element_type=jnp.float32)
    h1 = 1.0 / (1.0 + jnp.exp(-z1))
    g1 = lax.dot_general(h1, V0, (((1,), (1,)), ((), ())),
                         preferred_element_type=jnp.float32)
    q = q_ref[...]
    d = lax.dot_general(h1, q[:, 1:2], (((1,), (0,)), ((), ())),
                        preferred_element_type=jnp.float32)
    d3 = d.reshape(FBLK, S1, 1)

    z0 = lax.dot_general(c0_ref[:, :64], W, (((1,), (1,)), ((), ())),
                         preferred_element_type=jnp.float32)
    h0 = 1.0 / (1.0 + jnp.exp(-z0))
    gam = lax.dot_general(h0, q[:, 0:1], (((1,), (0,)), ((), ())),
                          preferred_element_type=jnp.float32)
    g3 = gam.reshape(FBLK, 1, 1)

    t = g3 + d3
    t = jnp.where(t >= 0.0, t, 0.01 * t)
    m = jnp.max(t, axis=1, keepdims=True)
    e = jnp.exp(t - m)
    bh = e / jnp.sum(e, axis=1, keepdims=True)
    bs_ref[...] = jnp.concatenate([bh, b1_ref[:, :, :DT]], axis=2)

    g13 = g1.reshape(FBLK, S1, 32)
    agg = jnp.sum(g13 * bh, axis=1)
    o1_ref[...] = agg.T


_fin_call = pl.pallas_call(
    _fin_body,
    grid=(B // FBLK,),
    in_specs=[
        pl.BlockSpec((FBLK, S1, 128), lambda i: (i, 0, 0)),
        pl.BlockSpec((FBLK, 128), lambda i: (i, 0)),
        pl.BlockSpec((FBLK, S1, 128), lambda i: (i, 0, 0)),
        pl.BlockSpec((32, 64), lambda i: (0, 0)),
        pl.BlockSpec((32, 32), lambda i: (0, 0)),
        pl.BlockSpec((32, 2), lambda i: (0, 0)),
    ],
    out_specs=[
        pl.BlockSpec((FBLK, S1, 1 + DT), lambda i: (i, 0, 0)),
        pl.BlockSpec((32, FBLK), lambda i: (0, i)),
    ],
    out_shape=[
        jax.ShapeDtypeStruct((B, S1, 1 + DT), jnp.float32),
        jax.ShapeDtypeStruct((32, B), jnp.float32),
    ],
)


def kernel(x, samples, V1_h0, w1_h0, V1_h1_att, w1_h1, V1_h1, weights_hops_1):
    a_col = V1_h1_att.T @ w1_h1[:32]
    b_col = V1_h1_att.T @ w1_h1[32:]
    M = jnp.concatenate(
        [V1_h1.T, a_col[:, None], b_col[:, None],
         jnp.zeros((ND, PD - 34), jnp.float32)], axis=1)
    q = jnp.stack([V1_h0.T @ w1_h0[:32], V1_h0.T @ w1_h0[32:]], axis=1)

    P = _proj_call(x, M)
    beta1, con1, con0 = _sc_call(P, samples)
    beta_step, out1 = _fin_call(con1, con0, beta1, weights_hops_1, V1_h0, q)
    return out1, beta_step


# speedup vs baseline: 1.2731x; 1.0082x over previous
import functools

import jax
import jax.numpy as jnp
from jax import lax
from jax.experimental import pallas as pl
from jax.experimental.pallas import tpu as pltpu
from jax.experimental.pallas import tpu_sc as plsc

N = 100000
ND = 128
B = 2048
S1 = 16
S2 = 256
DT = 16
PD = 128
NW = 32
BPW = B // NW
PBLK = 2000
FBLK = 256


def _proj_body(x_ref, m_ref, o_ref):
    o_ref[...] = lax.dot_general(
        x_ref[...], m_ref[...], (((1,), (0,)), ((), ())),
        preferred_element_type=jnp.float32)


_proj_call = pl.pallas_call(
    _proj_body,
    grid=(N // PBLK,),
    in_specs=[
        pl.BlockSpec((PBLK, ND), lambda i: (i, 0)),
        pl.BlockSpec((ND, PD), lambda i: (0, 0)),
    ],
    out_specs=pl.BlockSpec((PBLK, PD), lambda i: (i, 0)),
    out_shape=jax.ShapeDtypeStruct((N, PD), jnp.float32),
)


def _sc_body(p_hbm, samples_hbm, con1_hbm, con0_hbm,
             idx_v, rows_v, c1_v, c0_v, sem0, sem1, osem0, osem1):
    wid = lax.axis_index("s") * 2 + lax.axis_index("c")
    base = wid * BPW
    pltpu.sync_copy(samples_hbm.at[pl.ds(base, BPW)], idx_v)

    iota16 = lax.iota(jnp.int32, 16)
    c33 = jnp.full((16,), 33, jnp.int32)
    sems = (sem0, sem1)
    osems = (osem0, osem1)

    def softmax16(t):
        t = jnp.where(t >= 0.0, t, 0.01 * t)
        m = jnp.max(t)
        e = jnp.exp(t - m)
        return e / jnp.sum(e)

    def issue(i, slot):
        pltpu.async_copy(p_hbm.at[idx_v.at[i, pl.ds(0, 128)]],
                         rows_v.at[slot, pl.ds(0, 128)], sems[slot])
        pltpu.async_copy(p_hbm.at[idx_v.at[i, pl.ds(128, 128)]],
                         rows_v.at[slot, pl.ds(128, 128)], sems[slot])
        pltpu.async_copy(p_hbm.at[idx_v.at[i, pl.ds(256, 17)]],
                         rows_v.at[slot, pl.ds(256, 17)], sems[slot])

    def drain(slot):
        pltpu.make_async_copy(p_hbm.at[pl.ds(0, 273)],
                              rows_v.at[slot, pl.ds(0, 273)], sems[slot]).wait()

    def wait_out(slot):
        pltpu.make_async_copy(c0_v.at[slot], con0_hbm.at[base],
                              osems[slot]).wait()
        pltpu.make_async_copy(c1_v.at[slot], con1_hbm.at[base],
                              osems[slot]).wait()

    def compute(i, slot):
        rv = rows_v.at[slot]

        a0 = rv[0, pl.ds(32, 16)][0]
        bv = plsc.load_gather(rv, [1 + iota16, c33])
        b0 = softmax16(a0 + bv)
        acc_a = jnp.zeros((16,), jnp.float32)
        acc_b = jnp.zeros((16,), jnp.float32)
        for dd in range(16):
            w = b0[dd]
            acc_a = acc_a + w * rv[1 + dd, pl.ds(0, 16)]
            acc_b = acc_b + w * rv[1 + dd, pl.ds(16, 16)]
        c0_v[slot, pl.ds(0, 16)] = rv[0, pl.ds(0, 16)]
        c0_v[slot, pl.ds(16, 16)] = rv[0, pl.ds(16, 16)]
        c0_v[slot, pl.ds(32, 16)] = acc_a
        c0_v[slot, pl.ds(48, 16)] = acc_b

        def sbody(s, c):
            a_t = rv[1 + s, pl.ds(32, 16)][0]
            base_r = 17 + 16 * s
            bvs = plsc.load_gather(rv, [base_r + iota16, c33])
            bb = softmax16(a_t + bvs)
            c1_v[slot, s, pl.ds(64, 16)] = bb
            aa = jnp.zeros((16,), jnp.float32)
            ab = jnp.zeros((16,), jnp.float32)
            for dd in range(16):
                w = bb[dd]
                aa = aa + w * rv[base_r + dd, pl.ds(0, 16)]
                ab = ab + w * rv[base_r + dd, pl.ds(16, 16)]
            c1_v[slot, s, pl.ds(0, 16)] = rv[1 + s, pl.ds(0, 16)]
            c1_v[slot, s, pl.ds(16, 16)] = rv[1 + s, pl.ds(16, 16)]
            c1_v[slot, s, pl.ds(32, 16)] = aa
            c1_v[slot, s, pl.ds(48, 16)] = ab
            return c

        lax.fori_loop(0, S1, sbody, 0)

        pltpu.async_copy(c0_v.at[slot], con0_hbm.at[base + i], osems[slot])
        pltpu.async_copy(c1_v.at[slot], con1_hbm.at[base + i], osems[slot])

    issue(0, 0)

    def gbody(g, carry):
        i = 2 * g
        issue(i + 1, 1)
        drain(0)

        @pl.when(g > 0)
        def _():
            wait_out(0)

        compute(i, 0)

        @pl.when(i + 2 < BPW)
        def _():
            issue(i + 2, 0)

        drain(1)

        @pl.when(g > 0)
        def _():
            wait_out(1)

        compute(i + 1, 1)
        return carry

    lax.fori_loop(0, BPW // 2, gbody, 0)
    wait_out(0)
    wait_out(1)


_sc_call = functools.partial(
    pl.kernel,
    mesh=plsc.VectorSubcoreMesh(core_axis_name="c", subcore_axis_name="s"),
    compiler_params=pltpu.CompilerParams(
        needs_layout_passes=False, use_tc_tiling_on_sc=False),
    out_type=(
        jax.ShapeDtypeStruct((B, S1, 128), jnp.float32),
        jax.ShapeDtypeStruct((B, 128), jnp.float32),
    ),
    scratch_types=[
        pltpu.VMEM((BPW, 273), jnp.int32),
        pltpu.VMEM((2, 280, PD), jnp.float32),
        pltpu.VMEM((2, S1, 128), jnp.float32),
        pltpu.VMEM((2, 128), jnp.float32),
        pltpu.SemaphoreType.DMA,
        pltpu.SemaphoreType.DMA,
        pltpu.SemaphoreType.DMA,
        pltpu.SemaphoreType.DMA,
    ],
)(_sc_body)


def _fin_body(c1_ref, c0_ref, w_ref, v0_ref, q_ref, bs_ref, o1_ref):
    W = w_ref[...]
    V0 = v0_ref[...]

    c1f = c1_ref[:, :, :64].reshape(FBLK * S1, 64)
    z1 = lax.dot_general(c1f, W, (((1,), (1,)), ((), ())),
                         preferred_element_type=jnp.float32)
    h1 = 1.0 / (1.0 + jnp.exp(-z1))
    g1 = lax.dot_general(h1, V0, (((1,), (1,)), ((), ())),
                         preferred_element_type=jnp.float32)
    q = q_ref[...]
    d = lax.dot_general(h1, q[:, 1:2], (((1,), (0,)), ((), ())),
                        preferred_element_type=jnp.float32)
    d3 = d.reshape(FBLK, S1, 1)

    z0 = lax.dot_general(c0_ref[:, :64], W, (((1,), (1,)), ((), ())),
                         preferred_element_type=jnp.float32)
    h0 = 1.0 / (1.0 + jnp.exp(-z0))
    gam = lax.dot_general(h0, q[:, 0:1], (((1,), (0,)), ((), ())),
                          preferred_element_type=jnp.float32)
    g3 = gam.reshape(FBLK, 1, 1)

    t = g3 + d3
    t = jnp.where(t >= 0.0, t, 0.01 * t)
    m = jnp.max(t, axis=1, keepdims=True)
    e = jnp.exp(t - m)
    bh = e / jnp.sum(e, axis=1, keepdims=True)
    bs_ref[...] = jnp.concatenate([bh, c1_ref[:, :, 64:64 + DT]], axis=2)

    g13 = g1.reshape(FBLK, S1, 32)
    agg = jnp.sum(g13 * bh, axis=1)
    o1_ref[...] = agg.T


_fin_call = pl.pallas_call(
    _fin_body,
    grid=(B // FBLK,),
    in_specs=[
        pl.BlockSpec((FBLK, S1, 128), lambda i: (i, 0, 0)),
        pl.BlockSpec((FBLK, 128), lambda i: (i, 0)),
        pl.BlockSpec((32, 64), lambda i: (0, 0)),
        pl.BlockSpec((32, 32), lambda i: (0, 0)),
        pl.BlockSpec((32, 2), lambda i: (0, 0)),
    ],
    out_specs=[
        pl.BlockSpec((FBLK, S1, 1 + DT), lambda i: (i, 0, 0)),
        pl.BlockSpec((32, FBLK), lambda i: (0, i)),
    ],
    out_shape=[
        jax.ShapeDtypeStruct((B, S1, 1 + DT), jnp.float32),
        jax.ShapeDtypeStruct((32, B), jnp.float32),
    ],
)


def kernel(x, samples, V1_h0, w1_h0, V1_h1_att, w1_h1, V1_h1, weights_hops_1):
    a_col = V1_h1_att.T @ w1_h1[:32]
    b_col = V1_h1_att.T @ w1_h1[32:]
    M = jnp.concatenate(
        [V1_h1.T, a_col[:, None], b_col[:, None],
         jnp.zeros((ND, PD - 34), jnp.float32)], axis=1)
    q = jnp.stack([V1_h0.T @ w1_h0[:32], V1_h0.T @ w1_h0[32:]], axis=1)

    P = _proj_call(x, M)
    con1, con0 = _sc_call(P, samples)
    beta_step, out1 = _fin_call(con1, con0, weights_hops_1, V1_h0, q)
    return out1, beta_step


# speedup vs baseline: 1.3726x; 1.0781x over previous
import functools

import jax
import jax.numpy as jnp
from jax import lax
from jax.experimental import pallas as pl
from jax.experimental.pallas import tpu as pltpu
from jax.experimental.pallas import tpu_sc as plsc

N = 100000
ND = 128
B = 2048
S1 = 16
S2 = 256
DT = 16
PD = 128
NW = 32
BPW = B // NW
PBLK = 4000
FBLK = 256


def _proj_body(x_ref, m_ref, o_ref):
    o_ref[...] = lax.dot_general(
        x_ref[...], m_ref[...], (((1,), (0,)), ((), ())),
        preferred_element_type=jnp.float32)


_proj_call = pl.pallas_call(
    _proj_body,
    grid=(N // PBLK,),
    in_specs=[
        pl.BlockSpec((PBLK, ND), lambda i: (i, 0)),
        pl.BlockSpec((ND, PD), lambda i: (0, 0)),
    ],
    out_specs=pl.BlockSpec((PBLK, PD), lambda i: (i, 0)),
    out_shape=jax.ShapeDtypeStruct((N, PD), jnp.float32),
)


def _sc_body(p_hbm, samples_hbm, con1_hbm, con0_hbm,
             idx_v, rows_v, c1_v, c0_v, sem0, sem1, osem0, osem1):
    wid = lax.axis_index("s") * 2 + lax.axis_index("c")
    base = wid * BPW
    pltpu.sync_copy(samples_hbm.at[pl.ds(base, BPW)], idx_v)

    iota16 = lax.iota(jnp.int32, 16)
    c33 = jnp.full((16,), 33, jnp.int32)
    sems = (sem0, sem1)
    osems = (osem0, osem1)

    def softmax16(t):
        t = jnp.where(t >= 0.0, t, 0.01 * t)
        m = jnp.max(t)
        e = jnp.exp(t - m)
        return e / jnp.sum(e)

    def issue(i, slot):
        pltpu.async_copy(p_hbm.at[idx_v.at[i, pl.ds(0, 128)]],
                         rows_v.at[slot, pl.ds(0, 128)], sems[slot])
        pltpu.async_copy(p_hbm.at[idx_v.at[i, pl.ds(128, 128)]],
                         rows_v.at[slot, pl.ds(128, 128)], sems[slot])
        pltpu.async_copy(p_hbm.at[idx_v.at[i, pl.ds(256, 17)]],
                         rows_v.at[slot, pl.ds(256, 17)], sems[slot])

    def drain(slot):
        pltpu.make_async_copy(p_hbm.at[pl.ds(0, 273)],
                              rows_v.at[slot, pl.ds(0, 273)], sems[slot]).wait()

    def wait_out(slot):
        pltpu.make_async_copy(c0_v.at[slot], con0_hbm.at[base],
                              osems[slot]).wait()
        pltpu.make_async_copy(c1_v.at[slot], con1_hbm.at[base],
                              osems[slot]).wait()

    def compute(i, slot):
        rv = rows_v.at[slot]

        a0 = rv[0, pl.ds(32, 16)][0]
        bv = plsc.load_gather(rv, [1 + iota16, c33])
        b0 = softmax16(a0 + bv)
        acc_a = jnp.zeros((16,), jnp.float32)
        acc_b = jnp.zeros((16,), jnp.float32)
        for dd in range(16):
            w = b0[dd]
            acc_a = acc_a + w * rv[1 + dd, pl.ds(0, 16)]
            acc_b = acc_b + w * rv[1 + dd, pl.ds(16, 16)]
        c0_v[slot, pl.ds(0, 16)] = rv[0, pl.ds(0, 16)]
        c0_v[slot, pl.ds(16, 16)] = rv[0, pl.ds(16, 16)]
        c0_v[slot, pl.ds(32, 16)] = acc_a
        c0_v[slot, pl.ds(48, 16)] = acc_b

        def sbody(s, c):
            a_t = rv[1 + s, pl.ds(32, 16)][0]
            base_r = 17 + 16 * s
            bvs = plsc.load_gather(rv, [base_r + iota16, c33])
            bb = softmax16(a_t + bvs)
            c1_v[slot, s, pl.ds(64, 16)] = bb
            aa = jnp.zeros((16,), jnp.float32)
            ab = jnp.zeros((16,), jnp.float32)
            for dd in range(16):
                w = bb[dd]
                aa = aa + w * rv[base_r + dd, pl.ds(0, 16)]
                ab = ab + w * rv[base_r + dd, pl.ds(16, 16)]
            c1_v[slot, s, pl.ds(0, 16)] = rv[1 + s, pl.ds(0, 16)]
            c1_v[slot, s, pl.ds(16, 16)] = rv[1 + s, pl.ds(16, 16)]
            c1_v[slot, s, pl.ds(32, 16)] = aa
            c1_v[slot, s, pl.ds(48, 16)] = ab
            return c

        lax.fori_loop(0, S1, sbody, 0)

        pltpu.async_copy(c0_v.at[slot], con0_hbm.at[base + i], osems[slot])
        pltpu.async_copy(c1_v.at[slot], con1_hbm.at[base + i], osems[slot])

    issue(0, 0)

    def gbody(g, carry):
        i = 2 * g
        issue(i + 1, 1)
        drain(0)

        @pl.when(g > 0)
        def _():
            wait_out(0)

        compute(i, 0)

        @pl.when(i + 2 < BPW)
        def _():
            issue(i + 2, 0)

        drain(1)

        @pl.when(g > 0)
        def _():
            wait_out(1)

        compute(i + 1, 1)
        return carry

    lax.fori_loop(0, BPW // 2, gbody, 0)
    wait_out(0)
    wait_out(1)


_sc_call = functools.partial(
    pl.kernel,
    mesh=plsc.VectorSubcoreMesh(core_axis_name="c", subcore_axis_name="s"),
    compiler_params=pltpu.CompilerParams(
        needs_layout_passes=False, use_tc_tiling_on_sc=False),
    out_type=(
        jax.ShapeDtypeStruct((B, S1, 128), jnp.float32),
        jax.ShapeDtypeStruct((B, 128), jnp.float32),
    ),
    scratch_types=[
        pltpu.VMEM((BPW, 273), jnp.int32),
        pltpu.VMEM((2, 280, PD), jnp.float32),
        pltpu.VMEM((2, S1, 128), jnp.float32),
        pltpu.VMEM((2, 128), jnp.float32),
        pltpu.SemaphoreType.DMA,
        pltpu.SemaphoreType.DMA,
        pltpu.SemaphoreType.DMA,
        pltpu.SemaphoreType.DMA,
    ],
)(_sc_body)


def _fin_body(c1_ref, c0_ref, w_ref, v0_ref, q_ref, bs_ref, o1_ref):
    W = w_ref[...]
    V0 = v0_ref[...]

    c1f = c1_ref[:, :, :64].reshape(FBLK * S1, 64)
    z1 = lax.dot_general(c1f, W, (((1,), (1,)), ((), ())),
                         preferred_element_type=jnp.float32)
    h1 = 1.0 / (1.0 + jnp.exp(-z1))
    g1 = lax.dot_general(h1, V0, (((1,), (1,)), ((), ())),
                         preferred_element_type=jnp.float32)
    q = q_ref[...]
    d = lax.dot_general(h1, q[:, 1:2], (((1,), (0,)), ((), ())),
                        preferred_element_type=jnp.float32)
    d3 = d.reshape(FBLK, S1, 1)

    z0 = lax.dot_general(c0_ref[:, :64], W, (((1,), (1,)), ((), ())),
                         preferred_element_type=jnp.float32)
    h0 = 1.0 / (1.0 + jnp.exp(-z0))
    gam = lax.dot_general(h0, q[:, 0:1], (((1,), (0,)), ((), ())),
                          preferred_element_type=jnp.float32)
    g3 = gam.reshape(FBLK, 1, 1)

    t = g3 + d3
    t = jnp.where(t >= 0.0, t, 0.01 * t)
    m = jnp.max(t, axis=1, keepdims=True)
    e = jnp.exp(t - m)
    bh = e / jnp.sum(e, axis=1, keepdims=True)
    bs_ref[...] = jnp.concatenate([bh, c1_ref[:, :, 64:64 + DT]], axis=2)

    g13 = g1.reshape(FBLK, S1, 32)
    agg = jnp.sum(g13 * bh, axis=1)
    o1_ref[...] = agg.T


_fin_call = pl.pallas_call(
    _fin_body,
    grid=(B // FBLK,),
    in_specs=[
        pl.BlockSpec((FBLK, S1, 128), lambda i: (i, 0, 0)),
        pl.BlockSpec((FBLK, 128), lambda i: (i, 0)),
        pl.BlockSpec((32, 64), lambda i: (0, 0)),
        pl.BlockSpec((32, 32), lambda i: (0, 0)),
        pl.BlockSpec((32, 2), lambda i: (0, 0)),
    ],
    out_specs=[
        pl.BlockSpec((FBLK, S1, 1 + DT), lambda i: (i, 0, 0)),
        pl.BlockSpec((32, FBLK), lambda i: (0, i)),
    ],
    out_shape=[
        jax.ShapeDtypeStruct((B, S1, 1 + DT), jnp.float32),
        jax.ShapeDtypeStruct((32, B), jnp.float32),
    ],
)


def kernel(x, samples, V1_h0, w1_h0, V1_h1_att, w1_h1, V1_h1, weights_hops_1):
    a_col = V1_h1_att.T @ w1_h1[:32]
    b_col = V1_h1_att.T @ w1_h1[32:]
    M = jnp.concatenate(
        [V1_h1.T, a_col[:, None], b_col[:, None],
         jnp.zeros((ND, PD - 34), jnp.float32)], axis=1)
    q = jnp.stack([V1_h0.T @ w1_h0[:32], V1_h0.T @ w1_h0[32:]], axis=1)

    P = _proj_call(x, M)
    con1, con0 = _sc_call(P, samples)
    beta_step, out1 = _fin_call(con1, con0, weights_hops_1, V1_h0, q)
    return out1, beta_step


# speedup vs baseline: 1.4024x; 1.0217x over previous
import functools

import jax
import jax.numpy as jnp
from jax import lax
from jax.experimental import pallas as pl
from jax.experimental.pallas import tpu as pltpu
from jax.experimental.pallas import tpu_sc as plsc

N = 100000
ND = 128
B = 2048
S1 = 16
S2 = 256
DT = 16
PD = 128
NW = 32
BPW = B // NW
PBLK = 10000
FBLK = 256


def _proj_body(x_ref, m_ref, o_ref):
    o_ref[...] = lax.dot_general(
        x_ref[...], m_ref[...], (((1,), (0,)), ((), ())),
        preferred_element_type=jnp.float32)


_proj_call = pl.pallas_call(
    _proj_body,
    grid=(N // PBLK,),
    in_specs=[
        pl.BlockSpec((PBLK, ND), lambda i: (i, 0)),
        pl.BlockSpec((ND, PD), lambda i: (0, 0)),
    ],
    out_specs=pl.BlockSpec((PBLK, PD), lambda i: (i, 0)),
    out_shape=jax.ShapeDtypeStruct((N, PD), jnp.float32),
)


def _sc_body(p_hbm, samples_hbm, con1_hbm, con0_hbm,
             idx_v, rows_v, c1_v, c0_v, sem0, sem1, osem0, osem1):
    wid = lax.axis_index("s") * 2 + lax.axis_index("c")
    base = wid * BPW
    pltpu.sync_copy(samples_hbm.at[pl.ds(base, BPW)], idx_v)

    iota16 = lax.iota(jnp.int32, 16)
    c33 = jnp.full((16,), 33, jnp.int32)
    sems = (sem0, sem1)
    osems = (osem0, osem1)

    def softmax16(t):
        t = jnp.where(t >= 0.0, t, 0.01 * t)
        m = jnp.max(t)
        e = jnp.exp(t - m)
        return e / jnp.sum(e)

    def issue(i, slot):
        pltpu.async_copy(p_hbm.at[idx_v.at[i, pl.ds(0, 128)]],
                         rows_v.at[slot, pl.ds(0, 128)], sems[slot])
        pltpu.async_copy(p_hbm.at[idx_v.at[i, pl.ds(128, 128)]],
                         rows_v.at[slot, pl.ds(128, 128)], sems[slot])
        pltpu.async_copy(p_hbm.at[idx_v.at[i, pl.ds(256, 17)]],
                         rows_v.at[slot, pl.ds(256, 17)], sems[slot])

    def drain(slot):
        pltpu.make_async_copy(p_hbm.at[pl.ds(0, 273)],
                              rows_v.at[slot, pl.ds(0, 273)], sems[slot]).wait()

    def wait_out(slot):
        pltpu.make_async_copy(c0_v.at[slot], con0_hbm.at[base],
                              osems[slot]).wait()
        pltpu.make_async_copy(c1_v.at[slot], con1_hbm.at[base],
                              osems[slot]).wait()

    def compute(i, slot):
        rv = rows_v.at[slot]

        a0 = rv[0, pl.ds(32, 16)][0]
        bv = plsc.load_gather(rv, [1 + iota16, c33])
        b0 = softmax16(a0 + bv)
        acc_a = jnp.zeros((16,), jnp.float32)
        acc_b = jnp.zeros((16,), jnp.float32)
        for dd in range(16):
            w = b0[dd]
            acc_a = acc_a + w * rv[1 + dd, pl.ds(0, 16)]
            acc_b = acc_b + w * rv[1 + dd, pl.ds(16, 16)]
        c0_v[slot, pl.ds(0, 16)] = rv[0, pl.ds(0, 16)]
        c0_v[slot, pl.ds(16, 16)] = rv[0, pl.ds(16, 16)]
        c0_v[slot, pl.ds(32, 16)] = acc_a
        c0_v[slot, pl.ds(48, 16)] = acc_b

        def sbody(s, c):
            a_t = rv[1 + s, pl.ds(32, 16)][0]
            base_r = 17 + 16 * s
            bvs = plsc.load_gather(rv, [base_r + iota16, c33])
            bb = softmax16(a_t + bvs)
            c1_v[slot, s, pl.ds(64, 16)] = bb
            aa = jnp.zeros((16,), jnp.float32)
            ab = jnp.zeros((16,), jnp.float32)
            for dd in range(16):
                w = bb[dd]
                aa = aa + w * rv[base_r + dd, pl.ds(0, 16)]
                ab = ab + w * rv[base_r + dd, pl.ds(16, 16)]
            c1_v[slot, s, pl.ds(0, 16)] = rv[1 + s, pl.ds(0, 16)]
            c1_v[slot, s, pl.ds(16, 16)] = rv[1 + s, pl.ds(16, 16)]
            c1_v[slot, s, pl.ds(32, 16)] = aa
            c1_v[slot, s, pl.ds(48, 16)] = ab
            return c

        lax.fori_loop(0, S1, sbody, 0)

        pltpu.async_copy(c0_v.at[slot], con0_hbm.at[base + i], osems[slot])
        pltpu.async_copy(c1_v.at[slot], con1_hbm.at[base + i], osems[slot])

    issue(0, 0)

    def gbody(g, carry):
        i = 2 * g
        issue(i + 1, 1)
        drain(0)

        @pl.when(g > 0)
        def _():
            wait_out(0)

        compute(i, 0)

        @pl.when(i + 2 < BPW)
        def _():
            issue(i + 2, 0)

        drain(1)

        @pl.when(g > 0)
        def _():
            wait_out(1)

        compute(i + 1, 1)
        return carry

    lax.fori_loop(0, BPW // 2, gbody, 0)
    wait_out(0)
    wait_out(1)


_sc_call = functools.partial(
    pl.kernel,
    mesh=plsc.VectorSubcoreMesh(core_axis_name="c", subcore_axis_name="s"),
    compiler_params=pltpu.CompilerParams(
        needs_layout_passes=False, use_tc_tiling_on_sc=False),
    out_type=(
        jax.ShapeDtypeStruct((B, S1, 128), jnp.float32),
        jax.ShapeDtypeStruct((B, 128), jnp.float32),
    ),
    scratch_types=[
        pltpu.VMEM((BPW, 273), jnp.int32),
        pltpu.VMEM((2, 280, PD), jnp.float32),
        pltpu.VMEM((2, S1, 128), jnp.float32),
        pltpu.VMEM((2, 128), jnp.float32),
        pltpu.SemaphoreType.DMA,
        pltpu.SemaphoreType.DMA,
        pltpu.SemaphoreType.DMA,
        pltpu.SemaphoreType.DMA,
    ],
)(_sc_body)


def _fin_body(c1_ref, c0_ref, w_ref, v0_ref, q_ref, bs_ref, o1_ref):
    W = w_ref[...]
    V0 = v0_ref[...]

    c1f = c1_ref[:, :, :64].reshape(FBLK * S1, 64)
    z1 = lax.dot_general(c1f, W, (((1,), (1,)), ((), ())),
                         preferred_element_type=jnp.float32)
    h1 = 1.0 / (1.0 + jnp.exp(-z1))
    g1 = lax.dot_general(h1, V0, (((1,), (1,)), ((), ())),
                         preferred_element_type=jnp.float32)
    q = q_ref[...]
    d = lax.dot_general(h1, q[:, 1:2], (((1,), (0,)), ((), ())),
                        preferred_element_type=jnp.float32)
    d3 = d.reshape(FBLK, S1, 1)

    z0 = lax.dot_general(c0_ref[:, :64], W, (((1,), (1,)), ((), ())),
                         preferred_element_type=jnp.float32)
    h0 = 1.0 / (1.0 + jnp.exp(-z0))
    gam = lax.dot_general(h0, q[:, 0:1], (((1,), (0,)), ((), ())),
                          preferred_element_type=jnp.float32)
    g3 = gam.reshape(FBLK, 1, 1)

    t = g3 + d3
    t = jnp.where(t >= 0.0, t, 0.01 * t)
    m = jnp.max(t, axis=1, keepdims=True)
    e = jnp.exp(t - m)
    bh = e / jnp.sum(e, axis=1, keepdims=True)
    bs_ref[...] = jnp.concatenate([bh, c1_ref[:, :, 64:64 + DT]], axis=2)

    g13 = g1.reshape(FBLK, S1, 32)
    agg = jnp.sum(g13 * bh, axis=1)
    o1_ref[...] = agg.T


_fin_call = pl.pallas_call(
    _fin_body,
    grid=(B // FBLK,),
    in_specs=[
        pl.BlockSpec((FBLK, S1, 128), lambda i: (i, 0, 0)),
        pl.BlockSpec((FBLK, 128), lambda i: (i, 0)),
        pl.BlockSpec((32, 64), lambda i: (0, 0)),
        pl.BlockSpec((32, 32), lambda i: (0, 0)),
        pl.BlockSpec((32, 2), lambda i: (0, 0)),
    ],
    out_specs=[
        pl.BlockSpec((FBLK, S1, 1 + DT), lambda i: (i, 0, 0)),
        pl.BlockSpec((32, FBLK), lambda i: (0, i)),
    ],
    out_shape=[
        jax.ShapeDtypeStruct((B, S1, 1 + DT), jnp.float32),
        jax.ShapeDtypeStruct((32, B), jnp.float32),
    ],
)


def kernel(x, samples, V1_h0, w1_h0, V1_h1_att, w1_h1, V1_h1, weights_hops_1):
    a_col = V1_h1_att.T @ w1_h1[:32]
    b_col = V1_h1_att.T @ w1_h1[32:]
    M = jnp.concatenate(
        [V1_h1.T, a_col[:, None], b_col[:, None],
         jnp.zeros((ND, PD - 34), jnp.float32)], axis=1)
    q = jnp.stack([V1_h0.T @ w1_h0[:32], V1_h0.T @ w1_h0[32:]], axis=1)

    P = _proj_call(x, M)
    con1, con0 = _sc_call(P, samples)
    beta_step, out1 = _fin_call(con1, con0, weights_hops_1, V1_h0, q)
    return out1, beta_step


# speedup vs baseline: 1.4024x; 1.0000x over previous
import functools

import jax
import jax.numpy as jnp
from jax import lax
from jax.experimental import pallas as pl
from jax.experimental.pallas import tpu as pltpu
from jax.experimental.pallas import tpu_sc as plsc

N = 100000
ND = 128
B = 2048
S1 = 16
S2 = 256
DT = 16
PD = 128
NW = 32
BPW = B // NW
PBLK = 10000
FBLK = 256


def _proj_body(x_ref, m_ref, o_ref):
    o_ref[...] = lax.dot_general(
        x_ref[...], m_ref[...], (((1,), (0,)), ((), ())),
        preferred_element_type=jnp.float32)


_proj_call = pl.pallas_call(
    _proj_body,
    grid=(N // PBLK,),
    in_specs=[
        pl.BlockSpec((PBLK, ND), lambda i: (i, 0)),
        pl.BlockSpec((ND, PD), lambda i: (0, 0)),
    ],
    out_specs=pl.BlockSpec((PBLK, PD), lambda i: (i, 0)),
    out_shape=jax.ShapeDtypeStruct((N, PD), jnp.float32),
)


def _sc_body(p_hbm, samples_hbm, con1_hbm, con0_hbm,
             idx_v, rows_v, c1_v, c0_v, sem0, sem1, osem0, osem1):
    wid = lax.axis_index("s") * 2 + lax.axis_index("c")
    base = wid * BPW
    pltpu.sync_copy(samples_hbm.at[pl.ds(base, BPW)], idx_v)

    iota16 = lax.iota(jnp.int32, 16)
    c33 = jnp.full((16,), 33, jnp.int32)
    sems = (sem0, sem1)
    osems = (osem0, osem1)

    def softmax16(t):
        t = jnp.where(t >= 0.0, t, 0.01 * t)
        m = jnp.max(t)
        e = jnp.exp(t - m)
        return e / jnp.sum(e)

    def issue(i, slot):
        pltpu.async_copy(p_hbm.at[idx_v.at[i, pl.ds(0, 128)]],
                         rows_v.at[slot, pl.ds(0, 128)], sems[slot])
        pltpu.async_copy(p_hbm.at[idx_v.at[i, pl.ds(128, 128)]],
                         rows_v.at[slot, pl.ds(128, 128)], sems[slot])
        pltpu.async_copy(p_hbm.at[idx_v.at[i, pl.ds(256, 17)]],
                         rows_v.at[slot, pl.ds(256, 17)], sems[slot])

    def drain(slot):
        pltpu.make_async_copy(p_hbm.at[pl.ds(0, 273)],
                              rows_v.at[slot, pl.ds(0, 273)], sems[slot]).wait()

    def wait_out(slot):
        pltpu.make_async_copy(c0_v.at[slot], con0_hbm.at[base],
                              osems[slot]).wait()
        pltpu.make_async_copy(c1_v.at[slot], con1_hbm.at[base],
                              osems[slot]).wait()

    def compute(i, slot):
        rv = rows_v.at[slot]

        a0 = rv[0, pl.ds(32, 16)][0]
        bv = plsc.load_gather(rv, [1 + iota16, c33])
        b0 = softmax16(a0 + bv)
        pa = [jnp.zeros((16,), jnp.float32) for _ in range(4)]
        pb = [jnp.zeros((16,), jnp.float32) for _ in range(4)]
        for dd in range(16):
            w = b0[dd]
            pa[dd % 4] = pa[dd % 4] + w * rv[1 + dd, pl.ds(0, 16)]
            pb[dd % 4] = pb[dd % 4] + w * rv[1 + dd, pl.ds(16, 16)]
        acc_a = (pa[0] + pa[1]) + (pa[2] + pa[3])
        acc_b = (pb[0] + pb[1]) + (pb[2] + pb[3])
        c0_v[slot, pl.ds(0, 16)] = rv[0, pl.ds(0, 16)]
        c0_v[slot, pl.ds(16, 16)] = rv[0, pl.ds(16, 16)]
        c0_v[slot, pl.ds(32, 16)] = acc_a
        c0_v[slot, pl.ds(48, 16)] = acc_b

        def sbody(s, c):
            a_t = rv[1 + s, pl.ds(32, 16)][0]
            base_r = 17 + 16 * s
            bvs = plsc.load_gather(rv, [base_r + iota16, c33])
            bb = softmax16(a_t + bvs)
            c1_v[slot, s, pl.ds(64, 16)] = bb
            qa = [jnp.zeros((16,), jnp.float32) for _ in range(4)]
            qb = [jnp.zeros((16,), jnp.float32) for _ in range(4)]
            for dd in range(16):
                w = bb[dd]
                qa[dd % 4] = qa[dd % 4] + w * rv[base_r + dd, pl.ds(0, 16)]
                qb[dd % 4] = qb[dd % 4] + w * rv[base_r + dd, pl.ds(16, 16)]
            aa = (qa[0] + qa[1]) + (qa[2] + qa[3])
            ab = (qb[0] + qb[1]) + (qb[2] + qb[3])
            c1_v[slot, s, pl.ds(0, 16)] = rv[1 + s, pl.ds(0, 16)]
            c1_v[slot, s, pl.ds(16, 16)] = rv[1 + s, pl.ds(16, 16)]
            c1_v[slot, s, pl.ds(32, 16)] = aa
            c1_v[slot, s, pl.ds(48, 16)] = ab
            return c

        lax.fori_loop(0, S1, sbody, 0)

        pltpu.async_copy(c0_v.at[slot], con0_hbm.at[base + i], osems[slot])
        pltpu.async_copy(c1_v.at[slot], con1_hbm.at[base + i], osems[slot])

    issue(0, 0)

    def gbody(g, carry):
        i = 2 * g
        issue(i + 1, 1)
        drain(0)

        @pl.when(g > 0)
        def _():
            wait_out(0)

        compute(i, 0)

        @pl.when(i + 2 < BPW)
        def _():
            issue(i + 2, 0)

        drain(1)

        @pl.when(g > 0)
        def _():
            wait_out(1)

        compute(i + 1, 1)
        return carry

    lax.fori_loop(0, BPW // 2, gbody, 0)
    wait_out(0)
    wait_out(1)


_sc_call = functools.partial(
    pl.kernel,
    mesh=plsc.VectorSubcoreMesh(core_axis_name="c", subcore_axis_name="s"),
    compiler_params=pltpu.CompilerParams(
        needs_layout_passes=False, use_tc_tiling_on_sc=False),
    out_type=(
        jax.ShapeDtypeStruct((B, S1, 128), jnp.float32),
        jax.ShapeDtypeStruct((B, 128), jnp.float32),
    ),
    scratch_types=[
        pltpu.VMEM((BPW, 273), jnp.int32),
        pltpu.VMEM((2, 280, PD), jnp.float32),
        pltpu.VMEM((2, S1, 128), jnp.float32),
        pltpu.VMEM((2, 128), jnp.float32),
        pltpu.SemaphoreType.DMA,
        pltpu.SemaphoreType.DMA,
        pltpu.SemaphoreType.DMA,
        pltpu.SemaphoreType.DMA,
    ],
)(_sc_body)


def _fin_body(c1_ref, c0_ref, w_ref, v0_ref, q_ref, bs_ref, o1_ref):
    W = w_ref[...]
    V0 = v0_ref[...]

    c1f = c1_ref[:, :, :64].reshape(FBLK * S1, 64)
    z1 = lax.dot_general(c1f, W, (((1,), (1,)), ((), ())),
                         preferred_element_type=jnp.float32)
    h1 = 1.0 / (1.0 + jnp.exp(-z1))
    g1 = lax.dot_general(h1, V0, (((1,), (1,)), ((), ())),
                         preferred_element_type=jnp.float32)
    q = q_ref[...]
    d = lax.dot_general(h1, q[:, 1:2], (((1,), (0,)), ((), ())),
                        preferred_element_type=jnp.float32)
    d3 = d.reshape(FBLK, S1, 1)

    z0 = lax.dot_general(c0_ref[:, :64], W, (((1,), (1,)), ((), ())),
                         preferred_element_type=jnp.float32)
    h0 = 1.0 / (1.0 + jnp.exp(-z0))
    gam = lax.dot_general(h0, q[:, 0:1], (((1,), (0,)), ((), ())),
                          preferred_element_type=jnp.float32)
    g3 = gam.reshape(FBLK, 1, 1)

    t = g3 + d3
    t = jnp.where(t >= 0.0, t, 0.01 * t)
    m = jnp.max(t, axis=1, keepdims=True)
    e = jnp.exp(t - m)
    bh = e / jnp.sum(e, axis=1, keepdims=True)
    bs_ref[...] = jnp.concatenate([bh, c1_ref[:, :, 64:64 + DT]], axis=2)

    g13 = g1.reshape(FBLK, S1, 32)
    agg = jnp.sum(g13 * bh, axis=1)
    o1_ref[...] = agg.T


_fin_call = pl.pallas_call(
    _fin_body,
    grid=(B // FBLK,),
    in_specs=[
        pl.BlockSpec((FBLK, S1, 128), lambda i: (i, 0, 0)),
        pl.BlockSpec((FBLK, 128), lambda i: (i, 0)),
        pl.BlockSpec((32, 64), lambda i: (0, 0)),
        pl.BlockSpec((32, 32), lambda i: (0, 0)),
        pl.BlockSpec((32, 2), lambda i: (0, 0)),
    ],
    out_specs=[
        pl.BlockSpec((FBLK, S1, 1 + DT), lambda i: (i, 0, 0)),
        pl.BlockSpec((32, FBLK), lambda i: (0, i)),
    ],
    out_shape=[
        jax.ShapeDtypeStruct((B, S1, 1 + DT), jnp.float32),
        jax.ShapeDtypeStruct((32, B), jnp.float32),
    ],
)


def kernel(x, samples, V1_h0, w1_h0, V1_h1_att, w1_h1, V1_h1, weights_hops_1):
    a_col = V1_h1_att.T @ w1_h1[:32]
    b_col = V1_h1_att.T @ w1_h1[32:]
    M = jnp.concatenate(
        [V1_h1.T, a_col[:, None], b_col[:, None],
         jnp.zeros((ND, PD - 34), jnp.float32)], axis=1)
    q = jnp.stack([V1_h0.T @ w1_h0[:32], V1_h0.T @ w1_h0[32:]], axis=1)

    P = _proj_call(x, M)
    con1, con0 = _sc_call(P, samples)
    beta_step, out1 = _fin_call(con1, con0, weights_hops_1, V1_h0, q)
    return out1, beta_step


# speedup vs baseline: 1.4087x; 1.0045x over previous
import functools

import jax
import jax.numpy as jnp
from jax import lax
from jax.experimental import pallas as pl
from jax.experimental.pallas import tpu as pltpu
from jax.experimental.pallas import tpu_sc as plsc

N = 100000
ND = 128
B = 2048
S1 = 16
S2 = 256
DT = 16
PD = 128
NW = 32
BPW = B // NW
PBLK = 10000
FBLK = 512


def _proj_body(x_ref, m_ref, o_ref):
    o_ref[...] = lax.dot_general(
        x_ref[...], m_ref[...], (((1,), (0,)), ((), ())),
        preferred_element_type=jnp.float32)


_proj_call = pl.pallas_call(
    _proj_body,
    grid=(N // PBLK,),
    in_specs=[
        pl.BlockSpec((PBLK, ND), lambda i: (i, 0)),
        pl.BlockSpec((ND, PD), lambda i: (0, 0)),
    ],
    out_specs=pl.BlockSpec((PBLK, PD), lambda i: (i, 0)),
    out_shape=jax.ShapeDtypeStruct((N, PD), jnp.float32),
)


def _sc_body(p_hbm, samples_hbm, con1_hbm, con0_hbm,
             idx_v, rows_v, c1_v, c0_v, sem0, sem1, osem0, osem1):
    wid = lax.axis_index("s") * 2 + lax.axis_index("c")
    base = wid * BPW
    pltpu.sync_copy(samples_hbm.at[pl.ds(base, BPW)], idx_v)

    iota16 = lax.iota(jnp.int32, 16)
    c33 = jnp.full((16,), 33, jnp.int32)
    sems = (sem0, sem1)
    osems = (osem0, osem1)

    def softmax16(t):
        t = jnp.where(t >= 0.0, t, 0.01 * t)
        m = jnp.max(t)
        e = jnp.exp(t - m)
        return e / jnp.sum(e)

    def issue(i, slot):
        pltpu.async_copy(p_hbm.at[idx_v.at[i, pl.ds(0, 128)]],
                         rows_v.at[slot, pl.ds(0, 128)], sems[slot])
        pltpu.async_copy(p_hbm.at[idx_v.at[i, pl.ds(128, 128)]],
                         rows_v.at[slot, pl.ds(128, 128)], sems[slot])
        pltpu.async_copy(p_hbm.at[idx_v.at[i, pl.ds(256, 17)]],
                         rows_v.at[slot, pl.ds(256, 17)], sems[slot])

    def drain(slot):
        pltpu.make_async_copy(p_hbm.at[pl.ds(0, 273)],
                              rows_v.at[slot, pl.ds(0, 273)], sems[slot]).wait()

    def wait_out(slot):
        pltpu.make_async_copy(c0_v.at[slot], con0_hbm.at[base],
                              osems[slot]).wait()
        pltpu.make_async_copy(c1_v.at[slot], con1_hbm.at[base],
                              osems[slot]).wait()

    def compute(i, slot):
        rv = rows_v.at[slot]

        a0 = rv[0, pl.ds(32, 16)][0]
        bv = plsc.load_gather(rv, [1 + iota16, c33])
        b0 = softmax16(a0 + bv)
        pa = [jnp.zeros((16,), jnp.float32) for _ in range(4)]
        pb = [jnp.zeros((16,), jnp.float32) for _ in range(4)]
        for dd in range(16):
            w = b0[dd]
            pa[dd % 4] = pa[dd % 4] + w * rv[1 + dd, pl.ds(0, 16)]
            pb[dd % 4] = pb[dd % 4] + w * rv[1 + dd, pl.ds(16, 16)]
        acc_a = (pa[0] + pa[1]) + (pa[2] + pa[3])
        acc_b = (pb[0] + pb[1]) + (pb[2] + pb[3])
        c0_v[slot, pl.ds(0, 16)] = rv[0, pl.ds(0, 16)]
        c0_v[slot, pl.ds(16, 16)] = rv[0, pl.ds(16, 16)]
        c0_v[slot, pl.ds(32, 16)] = acc_a
        c0_v[slot, pl.ds(48, 16)] = acc_b

        def sbody(s, c):
            a_t = rv[1 + s, pl.ds(32, 16)][0]
            base_r = 17 + 16 * s
            bvs = plsc.load_gather(rv, [base_r + iota16, c33])
            bb = softmax16(a_t + bvs)
            c1_v[slot, s, pl.ds(64, 16)] = bb
            qa = [jnp.zeros((16,), jnp.float32) for _ in range(4)]
            qb = [jnp.zeros((16,), jnp.float32) for _ in range(4)]
            for dd in range(16):
                w = bb[dd]
                qa[dd % 4] = qa[dd % 4] + w * rv[base_r + dd, pl.ds(0, 16)]
                qb[dd % 4] = qb[dd % 4] + w * rv[base_r + dd, pl.ds(16, 16)]
            aa = (qa[0] + qa[1]) + (qa[2] + qa[3])
            ab = (qb[0] + qb[1]) + (qb[2] + qb[3])
            c1_v[slot, s, pl.ds(0, 16)] = rv[1 + s, pl.ds(0, 16)]
            c1_v[slot, s, pl.ds(16, 16)] = rv[1 + s, pl.ds(16, 16)]
            c1_v[slot, s, pl.ds(32, 16)] = aa
            c1_v[slot, s, pl.ds(48, 16)] = ab
            return c

        lax.fori_loop(0, S1, sbody, 0)

        pltpu.async_copy(c0_v.at[slot], con0_hbm.at[base + i], osems[slot])
        pltpu.async_copy(c1_v.at[slot], con1_hbm.at[base + i], osems[slot])

    issue(0, 0)

    def gbody(g, carry):
        i = 2 * g
        issue(i + 1, 1)
        drain(0)

        @pl.when(g > 0)
        def _():
            wait_out(0)

        compute(i, 0)

        @pl.when(i + 2 < BPW)
        def _():
            issue(i + 2, 0)

        drain(1)

        @pl.when(g > 0)
        def _():
            wait_out(1)

        compute(i + 1, 1)
        return carry

    lax.fori_loop(0, BPW // 2, gbody, 0)
    wait_out(0)
    wait_out(1)


_sc_call = functools.partial(
    pl.kernel,
    mesh=plsc.VectorSubcoreMesh(core_axis_name="c", subcore_axis_name="s"),
    compiler_params=pltpu.CompilerParams(
        needs_layout_passes=False, use_tc_tiling_on_sc=False),
    out_type=(
        jax.ShapeDtypeStruct((B, S1, 128), jnp.float32),
        jax.ShapeDtypeStruct((B, 128), jnp.float32),
    ),
    scratch_types=[
        pltpu.VMEM((BPW, 273), jnp.int32),
        pltpu.VMEM((2, 280, PD), jnp.float32),
        pltpu.VMEM((2, S1, 128), jnp.float32),
        pltpu.VMEM((2, 128), jnp.float32),
        pltpu.SemaphoreType.DMA,
        pltpu.SemaphoreType.DMA,
        pltpu.SemaphoreType.DMA,
        pltpu.SemaphoreType.DMA,
    ],
)(_sc_body)


def _fin_body(c1_ref, c0_ref, w_ref, v0_ref, q_ref, bs_ref, o1_ref):
    W = w_ref[...]
    V0 = v0_ref[...]

    c1f = c1_ref[:, :, :64].reshape(FBLK * S1, 64)
    z1 = lax.dot_general(c1f, W, (((1,), (1,)), ((), ())),
                         preferred_element_type=jnp.float32)
    h1 = 1.0 / (1.0 + jnp.exp(-z1))
    g1 = lax.dot_general(h1, V0, (((1,), (1,)), ((), ())),
                         preferred_element_type=jnp.float32)
    q = q_ref[...]
    d = lax.dot_general(h1, q[:, 1:2], (((1,), (0,)), ((), ())),
                        preferred_element_type=jnp.float32)
    d3 = d.reshape(FBLK, S1, 1)

    z0 = lax.dot_general(c0_ref[:, :64], W, (((1,), (1,)), ((), ())),
                         preferred_element_type=jnp.float32)
    h0 = 1.0 / (1.0 + jnp.exp(-z0))
    gam = lax.dot_general(h0, q[:, 0:1], (((1,), (0,)), ((), ())),
                          preferred_element_type=jnp.float32)
    g3 = gam.reshape(FBLK, 1, 1)

    t = g3 + d3
    t = jnp.where(t >= 0.0, t, 0.01 * t)
    m = jnp.max(t, axis=1, keepdims=True)
    e = jnp.exp(t - m)
    bh = e / jnp.sum(e, axis=1, keepdims=True)
    bs_ref[...] = jnp.concatenate([bh, c1_ref[:, :, 64:64 + DT]], axis=2)

    g13 = g1.reshape(FBLK, S1, 32)
    agg = jnp.sum(g13 * bh, axis=1)
    o1_ref[...] = agg.T


_fin_call = pl.pallas_call(
    _fin_body,
    grid=(B // FBLK,),
    in_specs=[
        pl.BlockSpec((FBLK, S1, 128), lambda i: (i, 0, 0)),
        pl.BlockSpec((FBLK, 128), lambda i: (i, 0)),
        pl.BlockSpec((32, 64), lambda i: (0, 0)),
        pl.BlockSpec((32, 32), lambda i: (0, 0)),
        pl.BlockSpec((32, 2), lambda i: (0, 0)),
    ],
    out_specs=[
        pl.BlockSpec((FBLK, S1, 1 + DT), lambda i: (i, 0, 0)),
        pl.BlockSpec((32, FBLK), lambda i: (0, i)),
    ],
    out_shape=[
        jax.ShapeDtypeStruct((B, S1, 1 + DT), jnp.float32),
        jax.ShapeDtypeStruct((32, B), jnp.float32),
    ],
)


def kernel(x, samples, V1_h0, w1_h0, V1_h1_att, w1_h1, V1_h1, weights_hops_1):
    a_col = V1_h1_att.T @ w1_h1[:32]
    b_col = V1_h1_att.T @ w1_h1[32:]
    M = jnp.concatenate(
        [V1_h1.T, a_col[:, None], b_col[:, None],
         jnp.zeros((ND, PD - 34), jnp.float32)], axis=1)
    q = jnp.stack([V1_h0.T @ w1_h0[:32], V1_h0.T @ w1_h0[32:]], axis=1)

    P = _proj_call(x, M)
    con1, con0 = _sc_call(P, samples)
    beta_step, out1 = _fin_call(con1, con0, weights_hops_1, V1_h0, q)
    return out1, beta_step
